# baseline (speedup 1.0000x reference)
import functools
import jax
import jax.numpy as jnp
from jax import lax
from jax.experimental import pallas as pl
from jax.experimental.pallas import tpu as pltpu
from jax.experimental.pallas import tpu_sc as plsc

N = 10000
E = 160000
F = 256
FH = 128
NC = 2
NS = 16
L = 16
K = 80
NCHUNK = E // K
DW = 128

def _mesh():
    return plsc.VectorSubcoreMesh(core_axis_name="c", subcore_axis_name="s",
                                  num_cores=NC, num_subcores=NS)


KC = 128
E2 = 163840
NCC = E2 // KC
EPT = E2 // (NC * NS)
CPP = EPT // KC


def _sc_prep_body(simd, srcd, dstd, simr, srcr, dstr,
                  ewd_hbm, ewr_hbm, degd_hbm, degr_hbm,
                  sb2, db2, ib, ew5, ewb, zdeg, deg_sh):
    c = lax.axis_index("c")
    s = lax.axis_index("s")
    wid = s * NC + c
    rpt = 1000
    zrows = 40

    @pl.loop(0, KC)
    def _ez(i):
        for f in range(DW // L):
            ewb[i, pl.ds(f * L, L)] = jnp.zeros((L,), jnp.float32)

    for sim, src2, dst2, ew_hbm, deg_hbm in (
            (simd, srcd, dstd, ewd_hbm, degd_hbm),
            (simr, srcr, dstr, ewr_hbm, degr_hbm)):
        @pl.loop(0, zrows)
        def _z(i):
            for f in range(DW // L):
                zdeg[i, pl.ds(f * L, L)] = jnp.zeros((L,), jnp.float32)

        @pl.when(s < 10)
        def _():
            @pl.loop(0, rpt // zrows)
            def _zs(j):
                pltpu.sync_copy(zdeg, deg_sh.at[pl.ds(s * rpt + j * zrows,
                                                      zrows)])
        plsc.subcore_barrier()

        pltpu.sync_copy(src2.at[pl.ds(wid * CPP, CPP)], sb2)
        pltpu.sync_copy(dst2.at[pl.ds(wid * CPP, CPP)], db2)

        @pl.loop(0, CPP)
        def _chunk(i):
            gidb = (wid * CPP + i) * KC
            for g in range(KC // L):
                sv = sb2[i, pl.ds(g * L, L)]
                dv = db2[i, pl.ds(g * L, L)]
                ib[pl.ds(g * L, L)] = sv * N + dv
            pltpu.sync_copy(sim.at[ib], ew5.at[pl.ds(i * KC, KC)])
            for g in range(KC // L):
                ids = lax.iota(jnp.int32, L) + (gidb + g * L)
                off = i * KC + g * L
                ew5[pl.ds(off, L)] = jnp.where(
                    ids < E, ew5[pl.ds(off, L)], 0.0)
            @pl.loop(0, KC // L)
            def _grp(g):
                ewv = ew5[pl.ds(i * KC + g * L, L)]
                ws = [jnp.take(ewv, jnp.zeros((L,), jnp.int32) + j)
                      for j in range(L)]
                lane0 = lax.iota(jnp.int32, L) == 0
                for j in range(L):
                    r = g * L + j
                    ewb[r, pl.ds(0, L)] = jnp.zeros((L,), jnp.float32)
                    ewb[r, pl.ds(0, L)] = ewb[r, pl.ds(0, L)] + jnp.where(
                        lane0, ws[j], 0.0)
            pltpu.sync_copy(ewb, deg_sh.at[db2.at[i]], add=True)

        pltpu.sync_copy(ew5, ew_hbm.at[pl.ds(wid * EPT, EPT)])
        plsc.subcore_barrier()

        @pl.when(s < 10)
        def _():
            @pl.loop(0, rpt // zrows)
            def _wb(j):
                pltpu.sync_copy(deg_sh.at[pl.ds(s * rpt + j * zrows,
                                                zrows)], zdeg)
                pltpu.sync_copy(zdeg,
                                deg_hbm.at[pl.ds(c * N + s * rpt
                                                 + j * zrows, zrows)])


def _sc_prep(simd, srcd2, dstd2, simr, srcr2, dstr2):
    kfn = pl.kernel(
        _sc_prep_body,
        out_type=[
            jax.ShapeDtypeStruct((E2,), jnp.float32),
            jax.ShapeDtypeStruct((E2,), jnp.float32),
            jax.ShapeDtypeStruct((NC * N, DW), jnp.float32),
            jax.ShapeDtypeStruct((NC * N, DW), jnp.float32),
        ],
        mesh=_mesh(),
        scratch_types=[
            pltpu.VMEM((CPP, KC), jnp.int32),
            pltpu.VMEM((CPP, KC), jnp.int32),
            pltpu.VMEM((KC,), jnp.int32),
            pltpu.VMEM((EPT,), jnp.float32),
            pltpu.VMEM((KC, DW), jnp.float32),
            pltpu.VMEM((40, DW), jnp.float32),
            pltpu.VMEM_SHARED((N, DW), jnp.float32),
        ],
    )
    return kfn(simd, srcd2, dstd2, simr, srcr2, dstr2)


CPT = NCC // NS
GP = 16


def _msg_one_graph(hp_hbm, src_hbm, dst_hbm, ew_hbm, acc_hbm,
                   sidx, didx, ewt, gbufs, zbuf, acc_sh, gsem, ssem, c, s):
    rpt = 1000
    zrows = 40

    @pl.loop(0, zrows)
    def _z(i):
        for f in range(FH // L):
            zbuf[i, pl.ds(f * L, L)] = jnp.zeros((L,), jnp.float32)

    @pl.when(s < 10)
    def _():
        @pl.loop(0, rpt // zrows)
        def _zs(j):
            pltpu.sync_copy(zbuf, acc_sh.at[pl.ds(s * rpt + j * zrows,
                                                  zrows)])
    plsc.subcore_barrier()

    def fire(k, b):
        pltpu.async_copy(hp_hbm.at[sidx.at[k]], gbufs[b], gsem)

    def wait_g(b):
        pltpu.make_async_copy(hp_hbm.at[pl.ds(0, KC)], gbufs[b], gsem).wait()

    def fire_s(k, b):
        pltpu.async_copy(gbufs[b], acc_sh.at[didx.at[k]], ssem, add=True)

    def wait_s(b):
        pltpu.make_async_copy(gbufs[b], acc_sh.at[pl.ds(0, KC)], ssem).wait()

    @pl.loop(0, CPT // GP)
    def _group(grp):
        gb = s * CPT + grp * GP
        pltpu.sync_copy(src_hbm.at[pl.ds(gb, GP)], sidx)
        pltpu.sync_copy(dst_hbm.at[pl.ds(gb, GP)], didx)
        pltpu.sync_copy(ew_hbm.at[pl.ds(gb, GP)], ewt)

        @pl.loop(0, GP)
        def _adj(r):
            for g in range(KC // L):
                sidx[r, pl.ds(g * L, L)] = sidx[r, pl.ds(g * L, L)] + c * N

        fire(0, 0)
        for k in range(GP):
            wait_g(k % 2)
            if k < GP - 1:
                if k >= 1:
                    wait_s((k + 1) % 2)
                fire(k + 1, (k + 1) % 2)
            g = gbufs[k % 2]

            @pl.loop(0, KC // L)
            def _scale(gi):
                ewv = ewt[k, pl.ds(gi * L, L)]
                ws = [jnp.take(ewv, jnp.zeros((L,), jnp.int32) + j)
                      for j in range(L)]
                for j in range(L):
                    e = gi * L + j
                    for f in range(FH // L):
                        g[e, pl.ds(f * L, L)] = (g[e, pl.ds(f * L, L)]
                                                 * ws[j])

            fire_s(k, k % 2)

        wait_s(0)
        wait_s(1)

    plsc.subcore_barrier()

    @pl.when(s < 10)
    def _():
        @pl.loop(0, rpt // zrows)
        def _wb(j):
            pltpu.sync_copy(acc_sh.at[pl.ds(s * rpt + j * zrows, zrows)],
                            zbuf)
            pltpu.sync_copy(zbuf,
                            acc_hbm.at[pl.ds(c * N + s * rpt + j * zrows,
                                             zrows)])


def _sc_msg_body(hpd, srcd, dstd, ewd, hpr, srcr, dstr, ewr,
                 accd_hbm, accr_hbm,
                 sidx, didx, ewt, gbuf0, gbuf1, zbuf, acc_sh, gsem, ssem):
    c = lax.axis_index("c")
    s = lax.axis_index("s")
    gbufs = (gbuf0, gbuf1)
    _msg_one_graph(hpd, srcd, dstd, ewd, accd_hbm, sidx, didx, ewt,
                   gbufs, zbuf, acc_sh, gsem, ssem, c, s)
    _msg_one_graph(hpr, srcr, dstr, ewr, accr_hbm, sidx, didx, ewt,
                   gbufs, zbuf, acc_sh, gsem, ssem, c, s)


def _sc_msg(hp_d, srcd2, dstd2, ewd2, hp_r, srcr2, dstr2, ewr2):
    kfn = pl.kernel(
        _sc_msg_body,
        out_type=[
            jax.ShapeDtypeStruct((NC * N, FH), jnp.float32),
            jax.ShapeDtypeStruct((NC * N, FH), jnp.float32),
        ],
        mesh=_mesh(),
        scratch_types=[
            pltpu.VMEM((GP, KC), jnp.int32),
            pltpu.VMEM((GP, KC), jnp.int32),
            pltpu.VMEM((GP, KC), jnp.float32),
            pltpu.VMEM((KC, FH), jnp.float32),
            pltpu.VMEM((KC, FH), jnp.float32),
            pltpu.VMEM((40, FH), jnp.float32),
            pltpu.VMEM_SHARED((N, FH), jnp.float32),
            pltpu.SemaphoreType.DMA,
            pltpu.SemaphoreType.DMA,
        ],
    )
    return kfn(hp_d, srcd2, dstd2, ewd2, hp_r, srcr2, dstr2, ewr2)


BM = 400


def _pre_body(x_ref, deg_ref, w_ref, o_ref):
    deg = deg_ref[...] + 1.0
    dinv = jnp.where(deg > 0, 1.0 / jnp.sqrt(deg), 0.0)
    o_ref[...] = jnp.dot(x_ref[...] * dinv, w_ref[...],
                         preferred_element_type=jnp.float32)


def _k_pre(x, deg_col, w):
    fin = x.shape[1]
    return pl.pallas_call(
        _pre_body,
        grid=(NC, N // BM),
        in_specs=[
            pl.BlockSpec((BM, fin), lambda c, i: (i, 0)),
            pl.BlockSpec((BM, 1), lambda c, i: (i, 0)),
            pl.BlockSpec((fin, FH), lambda c, i: (0, c)),
        ],
        out_specs=pl.BlockSpec((BM, FH), lambda c, i: (c * (N // BM) + i, 0)),
        out_shape=jax.ShapeDtypeStruct((NC * N, FH), jnp.float32),
    )(x, deg_col, w)


def _relu_combine(a0_ref, a1_ref, h0_ref, h1_ref, deg_ref, b_ref):
    deg = deg_ref[...] + 1.0
    dinv = jnp.where(deg > 0, 1.0 / jnp.sqrt(deg), 0.0)
    y0 = dinv * (a0_ref[...] + h0_ref[...])
    y1 = dinv * (a1_ref[...] + h1_ref[...])
    y = jnp.concatenate([y0, y1], axis=1) + b_ref[...]
    return jnp.maximum(y, 0.0), dinv


def _mid_body(a0_ref, a1_ref, h0_ref, h1_ref, deg_ref, b_ref, w_ref, o_ref):
    y, dinv = _relu_combine(a0_ref, a1_ref, h0_ref, h1_ref, deg_ref, b_ref)
    o_ref[...] = jnp.dot(y * dinv, w_ref[...],
                         preferred_element_type=jnp.float32)


def _k_mid(acc, hprime, deg_col, b, w):
    nb = N // BM
    return pl.pallas_call(
        _mid_body,
        grid=(NC, nb),
        in_specs=[
            pl.BlockSpec((BM, FH), lambda c, i: (i, 0)),
            pl.BlockSpec((BM, FH), lambda c, i: (nb + i, 0)),
            pl.BlockSpec((BM, FH), lambda c, i: (i, 0)),
            pl.BlockSpec((BM, FH), lambda c, i: (nb + i, 0)),
            pl.BlockSpec((BM, 1), lambda c, i: (i, 0)),
            pl.BlockSpec((1, F), lambda c, i: (0, 0)),
            pl.BlockSpec((F, FH), lambda c, i: (0, c)),
        ],
        out_specs=pl.BlockSpec((BM, FH), lambda c, i: (c * nb + i, 0)),
        out_shape=jax.ShapeDtypeStruct((NC * N, FH), jnp.float32),
    )(acc, acc, hprime, hprime, deg_col, b, w)


def _tail_body(a0_ref, a1_ref, h0_ref, h1_ref, deg_ref, b_ref,
               w1_ref, b1_ref, w2_ref, b2_ref, w3_ref, b3_ref, o_ref):
    y, _ = _relu_combine(a0_ref, a1_ref, h0_ref, h1_ref, deg_ref, b_ref)
    h = jnp.maximum(jnp.dot(y, w1_ref[...],
                            preferred_element_type=jnp.float32)
                    + b1_ref[...], 0.0)
    h = jnp.maximum(jnp.dot(h, w2_ref[...],
                            preferred_element_type=jnp.float32)
                    + b2_ref[...], 0.0)
    h = jnp.maximum(jnp.dot(h, w3_ref[...],
                            preferred_element_type=jnp.float32)
                    + b3_ref[...], 0.0)
    o_ref[...] = h


def _k_tail(acc, hprime, deg_col, b, w1, b1, w2, b2, w3, b3):
    nb = N // BM
    return pl.pallas_call(
        _tail_body,
        grid=(nb,),
        in_specs=[
            pl.BlockSpec((BM, FH), lambda i: (i, 0)),
            pl.BlockSpec((BM, FH), lambda i: (nb + i, 0)),
            pl.BlockSpec((BM, FH), lambda i: (i, 0)),
            pl.BlockSpec((BM, FH), lambda i: (nb + i, 0)),
            pl.BlockSpec((BM, 1), lambda i: (i, 0)),
            pl.BlockSpec((1, F), lambda i: (0, 0)),
            pl.BlockSpec((256, 256), lambda i: (0, 0)),
            pl.BlockSpec((1, 256), lambda i: (0, 0)),
            pl.BlockSpec((256, 128), lambda i: (0, 0)),
            pl.BlockSpec((1, 128), lambda i: (0, 0)),
            pl.BlockSpec((128, 64), lambda i: (0, 0)),
            pl.BlockSpec((1, 64), lambda i: (0, 0)),
        ],
        out_specs=pl.BlockSpec((BM, 64), lambda i: (i, 0)),
        out_shape=jax.ShapeDtypeStruct((N, 64), jnp.float32),
    )(acc, acc, hprime, hprime, deg_col, b, w1, b1, w2, b2, w3, b3)


BF = 400


def _final_body(a_ref, b_ref, o_ref):
    o_ref[...] = lax.dot_general(
        a_ref[...], b_ref[...],
        (((1,), (1,)), ((), ())),
        preferred_element_type=jnp.float32)


def _final(dis, drg):
    return pl.pallas_call(
        _final_body,
        grid=(N // BF,),
        in_specs=[
            pl.BlockSpec((BF, 64), lambda i: (i, 0)),
            pl.BlockSpec((N, 64), lambda i: (0, 0)),
        ],
        out_specs=pl.BlockSpec((BF, N), lambda i: (i, 0)),
        out_shape=jax.ShapeDtypeStruct((N, N), jnp.float32),
    )(dis, drg)


def kernel(drug_data, drug_edge_index, disease_data, disease_edge_index,
           disease_random, drug_random, Wg1d, bg1d, Wg2d, bg2d, Wg1r, bg1r,
           Wg2r, bg2r, Wl1d, bl1d, Wl2d, bl2d, Wl3d, bl3d, Wl1r, bl1r,
           Wl2r, bl2r, Wl3r, bl3r):
    ds_, dd_ = disease_edge_index[0], disease_edge_index[1]
    rs_, rd_ = drug_edge_index[0], drug_edge_index[1]

    zi = jnp.zeros((E2 - E,), jnp.int32)
    ds2 = jnp.concatenate([ds_, zi]).reshape(NCC, KC)
    dd2 = jnp.concatenate([dd_, zi]).reshape(NCC, KC)
    rs2 = jnp.concatenate([rs_, zi]).reshape(NCC, KC)
    rd2 = jnp.concatenate([rd_, zi]).reshape(NCC, KC)

    ew_d, ew_r, deg2_d, deg2_r = _sc_prep(
        disease_data.reshape(-1), ds2, dd2,
        drug_data.reshape(-1), rs2, rd2)

    degcol_d = (deg2_d[:N, :1] + deg2_d[N:, :1])
    degcol_r = (deg2_r[:N, :1] + deg2_r[N:, :1])
    ewd2 = ew_d.reshape(NCC, KC)
    ewr2 = ew_r.reshape(NCC, KC)

    hp_d = _k_pre(disease_random, degcol_d, Wg1d)
    hp_r = _k_pre(drug_random, degcol_r, Wg1r)
    acc_d, acc_r = _sc_msg(hp_d, ds2, dd2, ewd2, hp_r, rs2, rd2, ewr2)
    hp2_d = _k_mid(acc_d, hp_d, degcol_d, bg1d.reshape(1, F), Wg2d)
    hp2_r = _k_mid(acc_r, hp_r, degcol_r, bg1r.reshape(1, F), Wg2r)
    acc2_d, acc2_r = _sc_msg(hp2_d, ds2, dd2, ewd2,
                             hp2_r, rs2, rd2, ewr2)
    dis = _k_tail(acc2_d, hp2_d, degcol_d, bg2d.reshape(1, F),
                  Wl1d, bl1d.reshape(1, 256), Wl2d, bl2d.reshape(1, 128),
                  Wl3d, bl3d.reshape(1, 64))
    drg = _k_tail(acc2_r, hp2_r, degcol_r, bg2r.reshape(1, F),
                  Wl1r, bl1r.reshape(1, 256), Wl2r, bl2r.reshape(1, 128),
                  Wl3r, bl3r.reshape(1, 64))

    return _final(dis, drg)


# speedup vs baseline: 1.0226x; 1.0226x over previous
import functools
import jax
import jax.numpy as jnp
from jax import lax
from jax.experimental import pallas as pl
from jax.experimental.pallas import tpu as pltpu
from jax.experimental.pallas import tpu_sc as plsc

N = 10000
E = 160000
F = 256
FH = 128
NC = 2
NS = 16
L = 16
K = 80
NCHUNK = E // K
DW = 128

def _mesh():
    return plsc.VectorSubcoreMesh(core_axis_name="c", subcore_axis_name="s",
                                  num_cores=NC, num_subcores=NS)


KC = 128
E2 = 163840
NCC = E2 // KC
EPT = E2 // (NC * NS)
CPP = EPT // KC


def _sc_prep_body(simd, srcd, dstd, simr, srcr, dstr,
                  ewd_hbm, ewr_hbm, degd_hbm, degr_hbm,
                  sb2, db2, ib, ew5, ewb, zdeg, deg_sh):
    c = lax.axis_index("c")
    s = lax.axis_index("s")
    wid = s * NC + c
    rpt = 1000
    zrows = 40

    @pl.loop(0, KC)
    def _ez(i):
        for f in range(DW // L):
            ewb[i, pl.ds(f * L, L)] = jnp.zeros((L,), jnp.float32)

    for sim, src2, dst2, ew_hbm, deg_hbm in (
            (simd, srcd, dstd, ewd_hbm, degd_hbm),
            (simr, srcr, dstr, ewr_hbm, degr_hbm)):
        @pl.loop(0, zrows)
        def _z(i):
            for f in range(DW // L):
                zdeg[i, pl.ds(f * L, L)] = jnp.zeros((L,), jnp.float32)

        @pl.when(s < 10)
        def _():
            @pl.loop(0, rpt // zrows)
            def _zs(j):
                pltpu.sync_copy(zdeg, deg_sh.at[pl.ds(s * rpt + j * zrows,
                                                      zrows)])
        plsc.subcore_barrier()

        pltpu.sync_copy(src2.at[pl.ds(wid * CPP, CPP)], sb2)
        pltpu.sync_copy(dst2.at[pl.ds(wid * CPP, CPP)], db2)

        @pl.loop(0, CPP)
        def _chunk(i):
            gidb = (wid * CPP + i) * KC
            for g in range(KC // L):
                sv = sb2[i, pl.ds(g * L, L)]
                dv = db2[i, pl.ds(g * L, L)]
                ib[pl.ds(g * L, L)] = sv * N + dv
            pltpu.sync_copy(sim.at[ib], ew5.at[pl.ds(i * KC, KC)])
            for g in range(KC // L):
                ids = lax.iota(jnp.int32, L) + (gidb + g * L)
                off = i * KC + g * L
                ew5[pl.ds(off, L)] = jnp.where(
                    ids < E, ew5[pl.ds(off, L)], 0.0)
            @pl.loop(0, KC // L)
            def _grp(g):
                ewv = ew5[pl.ds(i * KC + g * L, L)]
                ws = [jnp.take(ewv, jnp.zeros((L,), jnp.int32) + j)
                      for j in range(L)]
                lane0 = lax.iota(jnp.int32, L) == 0
                for j in range(L):
                    r = g * L + j
                    ewb[r, pl.ds(0, L)] = jnp.zeros((L,), jnp.float32)
                    ewb[r, pl.ds(0, L)] = ewb[r, pl.ds(0, L)] + jnp.where(
                        lane0, ws[j], 0.0)
            pltpu.sync_copy(ewb, deg_sh.at[db2.at[i]], add=True)

        pltpu.sync_copy(ew5, ew_hbm.at[pl.ds(wid * EPT, EPT)])
        plsc.subcore_barrier()

        @pl.when(s < 10)
        def _():
            @pl.loop(0, rpt // zrows)
            def _wb(j):
                pltpu.sync_copy(deg_sh.at[pl.ds(s * rpt + j * zrows,
                                                zrows)], zdeg)
                pltpu.sync_copy(zdeg,
                                deg_hbm.at[pl.ds(c * N + s * rpt
                                                 + j * zrows, zrows)])


def _sc_prep(simd, srcd2, dstd2, simr, srcr2, dstr2):
    kfn = pl.kernel(
        _sc_prep_body,
        out_type=[
            jax.ShapeDtypeStruct((E2,), jnp.float32),
            jax.ShapeDtypeStruct((E2,), jnp.float32),
            jax.ShapeDtypeStruct((NC * N, DW), jnp.float32),
            jax.ShapeDtypeStruct((NC * N, DW), jnp.float32),
        ],
        mesh=_mesh(),
        scratch_types=[
            pltpu.VMEM((CPP, KC), jnp.int32),
            pltpu.VMEM((CPP, KC), jnp.int32),
            pltpu.VMEM((KC,), jnp.int32),
            pltpu.VMEM((EPT,), jnp.float32),
            pltpu.VMEM((KC, DW), jnp.float32),
            pltpu.VMEM((40, DW), jnp.float32),
            pltpu.VMEM_SHARED((N, DW), jnp.float32),
        ],
    )
    return kfn(simd, srcd2, dstd2, simr, srcr2, dstr2)


CPT = NCC // NS
GP = 16


def _msg_one_graph(hp_hbm, src_hbm, dst_hbm, ew_hbm, acc_hbm,
                   sidx, didx, ewt, gbufs, zbuf, acc_sh, gsem, ssem, c, s):
    rpt = 1000
    zrows = 40

    @pl.loop(0, zrows)
    def _z(i):
        for f in range(FH // L):
            zbuf[i, pl.ds(f * L, L)] = jnp.zeros((L,), jnp.float32)

    @pl.when(s < 10)
    def _():
        @pl.loop(0, rpt // zrows)
        def _zs(j):
            pltpu.sync_copy(zbuf, acc_sh.at[pl.ds(s * rpt + j * zrows,
                                                  zrows)])
    plsc.subcore_barrier()

    def fire(k, b):
        pltpu.async_copy(hp_hbm.at[sidx.at[k]], gbufs[b], gsem)

    def wait_g(b):
        pltpu.make_async_copy(hp_hbm.at[pl.ds(0, KC)], gbufs[b], gsem).wait()

    def fire_s(k, b):
        pltpu.async_copy(gbufs[b], acc_sh.at[didx.at[k]], ssem, add=True)

    def wait_s(b):
        pltpu.make_async_copy(gbufs[b], acc_sh.at[pl.ds(0, KC)], ssem).wait()

    @pl.loop(0, CPT // GP)
    def _group(grp):
        gb = s * CPT + grp * GP
        pltpu.sync_copy(src_hbm.at[pl.ds(gb, GP)], sidx)
        pltpu.sync_copy(dst_hbm.at[pl.ds(gb, GP)], didx)
        pltpu.sync_copy(ew_hbm.at[pl.ds(gb, GP)], ewt)

        @pl.loop(0, GP)
        def _adj(r):
            for g in range(KC // L):
                sidx[r, pl.ds(g * L, L)] = sidx[r, pl.ds(g * L, L)] + c * N

        fire(0, 0)
        for k in range(GP):
            wait_g(k % 2)
            if k < GP - 1:
                if k >= 1:
                    wait_s((k + 1) % 2)
                fire(k + 1, (k + 1) % 2)
            g = gbufs[k % 2]

            @pl.loop(0, KC // L)
            def _scale(gi):
                ewv = ewt[k, pl.ds(gi * L, L)]
                ws = [jnp.take(ewv, jnp.zeros((L,), jnp.int32) + j)
                      for j in range(L)]
                for j in range(L):
                    e = gi * L + j
                    for f in range(FH // L):
                        g[e, pl.ds(f * L, L)] = (g[e, pl.ds(f * L, L)]
                                                 * ws[j])

            fire_s(k, k % 2)

        wait_s(0)
        wait_s(1)

    plsc.subcore_barrier()

    @pl.when(s < 10)
    def _():
        @pl.loop(0, rpt // zrows)
        def _wb(j):
            pltpu.sync_copy(acc_sh.at[pl.ds(s * rpt + j * zrows, zrows)],
                            zbuf)
            pltpu.sync_copy(zbuf,
                            acc_hbm.at[pl.ds(c * N + s * rpt + j * zrows,
                                             zrows)])


def _sc_msg_body(hp_hbm, src_hbm, dst_hbm, ew_hbm, acc_hbm,
                 sidx, didx, ewt, gbuf0, gbuf1, zbuf, acc_sh, gsem, ssem):
    c = lax.axis_index("c")
    s = lax.axis_index("s")
    gbufs = (gbuf0, gbuf1)
    _msg_one_graph(hp_hbm, src_hbm, dst_hbm, ew_hbm, acc_hbm, sidx, didx,
                   ewt, gbufs, zbuf, acc_sh, gsem, ssem, c, s)


def _sc_msg(hprime, src2d, dst2d, ew2d):
    kfn = pl.kernel(
        _sc_msg_body,
        out_type=jax.ShapeDtypeStruct((NC * N, FH), jnp.float32),
        mesh=_mesh(),
        scratch_types=[
            pltpu.VMEM((GP, KC), jnp.int32),
            pltpu.VMEM((GP, KC), jnp.int32),
            pltpu.VMEM((GP, KC), jnp.float32),
            pltpu.VMEM((KC, FH), jnp.float32),
            pltpu.VMEM((KC, FH), jnp.float32),
            pltpu.VMEM((40, FH), jnp.float32),
            pltpu.VMEM_SHARED((N, FH), jnp.float32),
            pltpu.SemaphoreType.DMA,
            pltpu.SemaphoreType.DMA,
        ],
    )
    return kfn(hprime, src2d, dst2d, ew2d)


BM = 400


def _pre_body(x_ref, deg_ref, w_ref, o_ref):
    deg = deg_ref[...] + 1.0
    dinv = jnp.where(deg > 0, 1.0 / jnp.sqrt(deg), 0.0)
    o_ref[...] = jnp.dot(x_ref[...] * dinv, w_ref[...],
                         preferred_element_type=jnp.float32)


def _k_pre(x, deg_col, w):
    fin = x.shape[1]
    return pl.pallas_call(
        _pre_body,
        grid=(NC, N // BM),
        in_specs=[
            pl.BlockSpec((BM, fin), lambda c, i: (i, 0)),
            pl.BlockSpec((BM, 1), lambda c, i: (i, 0)),
            pl.BlockSpec((fin, FH), lambda c, i: (0, c)),
        ],
        out_specs=pl.BlockSpec((BM, FH), lambda c, i: (c * (N // BM) + i, 0)),
        out_shape=jax.ShapeDtypeStruct((NC * N, FH), jnp.float32),
    )(x, deg_col, w)


def _relu_combine(a0_ref, a1_ref, h0_ref, h1_ref, deg_ref, b_ref):
    deg = deg_ref[...] + 1.0
    dinv = jnp.where(deg > 0, 1.0 / jnp.sqrt(deg), 0.0)
    y0 = dinv * (a0_ref[...] + h0_ref[...])
    y1 = dinv * (a1_ref[...] + h1_ref[...])
    y = jnp.concatenate([y0, y1], axis=1) + b_ref[...]
    return jnp.maximum(y, 0.0), dinv


def _mid_body(a0_ref, a1_ref, h0_ref, h1_ref, deg_ref, b_ref, w_ref, o_ref):
    y, dinv = _relu_combine(a0_ref, a1_ref, h0_ref, h1_ref, deg_ref, b_ref)
    o_ref[...] = jnp.dot(y * dinv, w_ref[...],
                         preferred_element_type=jnp.float32)


def _k_mid(acc, hprime, deg_col, b, w):
    nb = N // BM
    return pl.pallas_call(
        _mid_body,
        grid=(NC, nb),
        in_specs=[
            pl.BlockSpec((BM, FH), lambda c, i: (i, 0)),
            pl.BlockSpec((BM, FH), lambda c, i: (nb + i, 0)),
            pl.BlockSpec((BM, FH), lambda c, i: (i, 0)),
            pl.BlockSpec((BM, FH), lambda c, i: (nb + i, 0)),
            pl.BlockSpec((BM, 1), lambda c, i: (i, 0)),
            pl.BlockSpec((1, F), lambda c, i: (0, 0)),
            pl.BlockSpec((F, FH), lambda c, i: (0, c)),
        ],
        out_specs=pl.BlockSpec((BM, FH), lambda c, i: (c * nb + i, 0)),
        out_shape=jax.ShapeDtypeStruct((NC * N, FH), jnp.float32),
    )(acc, acc, hprime, hprime, deg_col, b, w)


def _tail_body(a0_ref, a1_ref, h0_ref, h1_ref, deg_ref, b_ref,
               w1_ref, b1_ref, w2_ref, b2_ref, w3_ref, b3_ref, o_ref):
    y, _ = _relu_combine(a0_ref, a1_ref, h0_ref, h1_ref, deg_ref, b_ref)
    h = jnp.maximum(jnp.dot(y, w1_ref[...],
                            preferred_element_type=jnp.float32)
                    + b1_ref[...], 0.0)
    h = jnp.maximum(jnp.dot(h, w2_ref[...],
                            preferred_element_type=jnp.float32)
                    + b2_ref[...], 0.0)
    h = jnp.maximum(jnp.dot(h, w3_ref[...],
                            preferred_element_type=jnp.float32)
                    + b3_ref[...], 0.0)
    o_ref[...] = h


def _k_tail(acc, hprime, deg_col, b, w1, b1, w2, b2, w3, b3):
    nb = N // BM
    return pl.pallas_call(
        _tail_body,
        grid=(nb,),
        in_specs=[
            pl.BlockSpec((BM, FH), lambda i: (i, 0)),
            pl.BlockSpec((BM, FH), lambda i: (nb + i, 0)),
            pl.BlockSpec((BM, FH), lambda i: (i, 0)),
            pl.BlockSpec((BM, FH), lambda i: (nb + i, 0)),
            pl.BlockSpec((BM, 1), lambda i: (i, 0)),
            pl.BlockSpec((1, F), lambda i: (0, 0)),
            pl.BlockSpec((256, 256), lambda i: (0, 0)),
            pl.BlockSpec((1, 256), lambda i: (0, 0)),
            pl.BlockSpec((256, 128), lambda i: (0, 0)),
            pl.BlockSpec((1, 128), lambda i: (0, 0)),
            pl.BlockSpec((128, 64), lambda i: (0, 0)),
            pl.BlockSpec((1, 64), lambda i: (0, 0)),
        ],
        out_specs=pl.BlockSpec((BM, 64), lambda i: (i, 0)),
        out_shape=jax.ShapeDtypeStruct((N, 64), jnp.float32),
    )(acc, acc, hprime, hprime, deg_col, b, w1, b1, w2, b2, w3, b3)


BF = 400


def _final_body(a_ref, b_ref, o_ref):
    o_ref[...] = lax.dot_general(
        a_ref[...], b_ref[...],
        (((1,), (1,)), ((), ())),
        preferred_element_type=jnp.float32)


def _final(dis, drg):
    return pl.pallas_call(
        _final_body,
        grid=(N // BF,),
        in_specs=[
            pl.BlockSpec((BF, 64), lambda i: (i, 0)),
            pl.BlockSpec((N, 64), lambda i: (0, 0)),
        ],
        out_specs=pl.BlockSpec((BF, N), lambda i: (i, 0)),
        out_shape=jax.ShapeDtypeStruct((N, N), jnp.float32),
    )(dis, drg)


def kernel(drug_data, drug_edge_index, disease_data, disease_edge_index,
           disease_random, drug_random, Wg1d, bg1d, Wg2d, bg2d, Wg1r, bg1r,
           Wg2r, bg2r, Wl1d, bl1d, Wl2d, bl2d, Wl3d, bl3d, Wl1r, bl1r,
           Wl2r, bl2r, Wl3r, bl3r):
    ds_, dd_ = disease_edge_index[0], disease_edge_index[1]
    rs_, rd_ = drug_edge_index[0], drug_edge_index[1]

    zi = jnp.zeros((E2 - E,), jnp.int32)
    ds2 = jnp.concatenate([ds_, zi]).reshape(NCC, KC)
    dd2 = jnp.concatenate([dd_, zi]).reshape(NCC, KC)
    rs2 = jnp.concatenate([rs_, zi]).reshape(NCC, KC)
    rd2 = jnp.concatenate([rd_, zi]).reshape(NCC, KC)

    ew_d, ew_r, deg2_d, deg2_r = _sc_prep(
        disease_data.reshape(-1), ds2, dd2,
        drug_data.reshape(-1), rs2, rd2)

    degcol_d = (deg2_d[:N, :1] + deg2_d[N:, :1])
    degcol_r = (deg2_r[:N, :1] + deg2_r[N:, :1])
    ewd2 = ew_d.reshape(NCC, KC)
    ewr2 = ew_r.reshape(NCC, KC)

    hp_d = _k_pre(disease_random, degcol_d, Wg1d)
    hp_r = _k_pre(drug_random, degcol_r, Wg1r)
    acc_d = _sc_msg(hp_d, ds2, dd2, ewd2)
    acc_r = _sc_msg(hp_r, rs2, rd2, ewr2)
    hp2_d = _k_mid(acc_d, hp_d, degcol_d, bg1d.reshape(1, F), Wg2d)
    hp2_r = _k_mid(acc_r, hp_r, degcol_r, bg1r.reshape(1, F), Wg2r)
    acc2_d = _sc_msg(hp2_d, ds2, dd2, ewd2)
    acc2_r = _sc_msg(hp2_r, rs2, rd2, ewr2)
    dis = _k_tail(acc2_d, hp2_d, degcol_d, bg2d.reshape(1, F),
                  Wl1d, bl1d.reshape(1, 256), Wl2d, bl2d.reshape(1, 128),
                  Wl3d, bl3d.reshape(1, 64))
    drg = _k_tail(acc2_r, hp2_r, degcol_r, bg2r.reshape(1, F),
                  Wl1r, bl1r.reshape(1, 256), Wl2r, bl2r.reshape(1, 128),
                  Wl3r, bl3r.reshape(1, 64))

    return _final(dis, drg)


# speedup vs baseline: 1.0898x; 1.0657x over previous
import functools
import jax
import jax.numpy as jnp
from jax import lax
from jax.experimental import pallas as pl
from jax.experimental.pallas import tpu as pltpu
from jax.experimental.pallas import tpu_sc as plsc

N = 10000
E = 160000
F = 256
FH = 128
NC = 2
NS = 16
L = 16
K = 80
NCHUNK = E // K
DW = 128

def _mesh():
    return plsc.VectorSubcoreMesh(core_axis_name="c", subcore_axis_name="s",
                                  num_cores=NC, num_subcores=NS)


KC = 128
E2 = 163840
NCC = E2 // KC
EPT = E2 // (NC * NS)
CPP = EPT // KC


def _sc_prep_body(sim, src2, dst2, ew_hbm, deg_hbm,
                  sb2, db2, ib, ew5, ewb, zdeg, deg_sh):
    c = lax.axis_index("c")
    s = lax.axis_index("s")
    wid = s * NC + c
    rpt = 1000
    zrows = 40

    @pl.loop(0, KC)
    def _ez(i):
        for f in range(DW // L):
            ewb[i, pl.ds(f * L, L)] = jnp.zeros((L,), jnp.float32)

    @pl.loop(0, zrows)
    def _z(i):
        for f in range(DW // L):
            zdeg[i, pl.ds(f * L, L)] = jnp.zeros((L,), jnp.float32)

    @pl.when(s < 10)
    def _():
        @pl.loop(0, rpt // zrows)
        def _zs(j):
            pltpu.sync_copy(zdeg, deg_sh.at[pl.ds(s * rpt + j * zrows,
                                                  zrows)])
    plsc.subcore_barrier()

    pltpu.sync_copy(src2.at[pl.ds(wid * CPP, CPP)], sb2)
    pltpu.sync_copy(dst2.at[pl.ds(wid * CPP, CPP)], db2)

    @pl.loop(0, CPP)
    def _chunk(i):
        gidb = (wid * CPP + i) * KC
        for g in range(KC // L):
            sv = sb2[i, pl.ds(g * L, L)]
            dv = db2[i, pl.ds(g * L, L)]
            ib[pl.ds(g * L, L)] = sv * N + dv
        pltpu.sync_copy(sim.at[ib], ew5.at[pl.ds(i * KC, KC)])
        for g in range(KC // L):
            ids = lax.iota(jnp.int32, L) + (gidb + g * L)
            off = i * KC + g * L
            ew5[pl.ds(off, L)] = jnp.where(
                ids < E, ew5[pl.ds(off, L)], 0.0)
        @pl.loop(0, KC // L)
        def _grp(g):
            ewv = ew5[pl.ds(i * KC + g * L, L)]
            ws = [jnp.take(ewv, jnp.zeros((L,), jnp.int32) + j)
                  for j in range(L)]
            lane0 = lax.iota(jnp.int32, L) == 0
            for j in range(L):
                r = g * L + j
                ewb[r, pl.ds(0, L)] = jnp.zeros((L,), jnp.float32)
                ewb[r, pl.ds(0, L)] = ewb[r, pl.ds(0, L)] + jnp.where(
                    lane0, ws[j], 0.0)
        pltpu.sync_copy(ewb, deg_sh.at[db2.at[i]], add=True)

    pltpu.sync_copy(ew5, ew_hbm.at[pl.ds(wid * EPT, EPT)])
    plsc.subcore_barrier()

    @pl.when(s < 10)
    def _():
        @pl.loop(0, rpt // zrows)
        def _wb(j):
            pltpu.sync_copy(deg_sh.at[pl.ds(s * rpt + j * zrows,
                                            zrows)], zdeg)
            pltpu.sync_copy(zdeg,
                            deg_hbm.at[pl.ds(c * N + s * rpt
                                             + j * zrows, zrows)])


def _sc_prep(sim, src2, dst2):
    kfn = pl.kernel(
        _sc_prep_body,
        out_type=[
            jax.ShapeDtypeStruct((E2,), jnp.float32),
            jax.ShapeDtypeStruct((NC * N, DW), jnp.float32),
        ],
        mesh=_mesh(),
        scratch_types=[
            pltpu.VMEM((CPP, KC), jnp.int32),
            pltpu.VMEM((CPP, KC), jnp.int32),
            pltpu.VMEM((KC,), jnp.int32),
            pltpu.VMEM((EPT,), jnp.float32),
            pltpu.VMEM((KC, DW), jnp.float32),
            pltpu.VMEM((40, DW), jnp.float32),
            pltpu.VMEM_SHARED((N, DW), jnp.float32),
        ],
    )
    return kfn(sim, src2, dst2)


CPT = NCC // NS
GP = 16


def _msg_one_graph(hp_hbm, src_hbm, dst_hbm, ew_hbm, acc_hbm,
                   sidx, didx, ewt, gbufs, zbuf, acc_sh, gsem, ssem, c, s):
    rpt = 1000
    zrows = 40

    @pl.loop(0, zrows)
    def _z(i):
        for f in range(FH // L):
            zbuf[i, pl.ds(f * L, L)] = jnp.zeros((L,), jnp.float32)

    @pl.when(s < 10)
    def _():
        @pl.loop(0, rpt // zrows)
        def _zs(j):
            pltpu.sync_copy(zbuf, acc_sh.at[pl.ds(s * rpt + j * zrows,
                                                  zrows)])
    plsc.subcore_barrier()

    def fire(k, b):
        pltpu.async_copy(hp_hbm.at[sidx.at[k]], gbufs[b], gsem)

    def wait_g(b):
        pltpu.make_async_copy(hp_hbm.at[pl.ds(0, KC)], gbufs[b], gsem).wait()

    def fire_s(k, b):
        pltpu.async_copy(gbufs[b], acc_sh.at[didx.at[k]], ssem, add=True)

    def wait_s(b):
        pltpu.make_async_copy(gbufs[b], acc_sh.at[pl.ds(0, KC)], ssem).wait()

    @pl.loop(0, CPT // GP)
    def _group(grp):
        gb = s * CPT + grp * GP
        pltpu.sync_copy(src_hbm.at[pl.ds(gb, GP)], sidx)
        pltpu.sync_copy(dst_hbm.at[pl.ds(gb, GP)], didx)
        pltpu.sync_copy(ew_hbm.at[pl.ds(gb, GP)], ewt)

        @pl.loop(0, GP)
        def _adj(r):
            for g in range(KC // L):
                sidx[r, pl.ds(g * L, L)] = sidx[r, pl.ds(g * L, L)] + c * N

        fire(0, 0)
        for k in range(GP):
            wait_g(k % 2)
            if k < GP - 1:
                if k >= 1:
                    wait_s((k + 1) % 2)
                fire(k + 1, (k + 1) % 2)
            g = gbufs[k % 2]

            @pl.loop(0, KC // L)
            def _scale(gi):
                ewv = ewt[k, pl.ds(gi * L, L)]
                ws = [jnp.take(ewv, jnp.zeros((L,), jnp.int32) + j)
                      for j in range(L)]
                for j in range(L):
                    e = gi * L + j
                    for f in range(FH // L):
                        g[e, pl.ds(f * L, L)] = (g[e, pl.ds(f * L, L)]
                                                 * ws[j])

            fire_s(k, k % 2)

        wait_s(0)
        wait_s(1)

    plsc.subcore_barrier()

    @pl.when(s < 10)
    def _():
        @pl.loop(0, rpt // zrows)
        def _wb(j):
            pltpu.sync_copy(acc_sh.at[pl.ds(s * rpt + j * zrows, zrows)],
                            zbuf)
            pltpu.sync_copy(zbuf,
                            acc_hbm.at[pl.ds(c * N + s * rpt + j * zrows,
                                             zrows)])


def _sc_msg_body(hp_hbm, src_hbm, dst_hbm, ew_hbm, acc_hbm,
                 sidx, didx, ewt, gbuf0, gbuf1, zbuf, acc_sh, gsem, ssem):
    c = lax.axis_index("c")
    s = lax.axis_index("s")
    gbufs = (gbuf0, gbuf1)
    _msg_one_graph(hp_hbm, src_hbm, dst_hbm, ew_hbm, acc_hbm, sidx, didx,
                   ewt, gbufs, zbuf, acc_sh, gsem, ssem, c, s)


def _sc_msg(hprime, src2d, dst2d, ew2d):
    kfn = pl.kernel(
        _sc_msg_body,
        out_type=jax.ShapeDtypeStruct((NC * N, FH), jnp.float32),
        mesh=_mesh(),
        scratch_types=[
            pltpu.VMEM((GP, KC), jnp.int32),
            pltpu.VMEM((GP, KC), jnp.int32),
            pltpu.VMEM((GP, KC), jnp.float32),
            pltpu.VMEM((KC, FH), jnp.float32),
            pltpu.VMEM((KC, FH), jnp.float32),
            pltpu.VMEM((40, FH), jnp.float32),
            pltpu.VMEM_SHARED((N, FH), jnp.float32),
            pltpu.SemaphoreType.DMA,
            pltpu.SemaphoreType.DMA,
        ],
    )
    return kfn(hprime, src2d, dst2d, ew2d)


BM = 400


def _pre_body(x_ref, deg_ref, w_ref, o_ref):
    deg = deg_ref[...] + 1.0
    dinv = jnp.where(deg > 0, 1.0 / jnp.sqrt(deg), 0.0)
    o_ref[...] = jnp.dot(x_ref[...] * dinv, w_ref[...],
                         preferred_element_type=jnp.float32)


def _k_pre(x, deg_col, w):
    fin = x.shape[1]
    return pl.pallas_call(
        _pre_body,
        grid=(NC, N // BM),
        in_specs=[
            pl.BlockSpec((BM, fin), lambda c, i: (i, 0)),
            pl.BlockSpec((BM, 1), lambda c, i: (i, 0)),
            pl.BlockSpec((fin, FH), lambda c, i: (0, c)),
        ],
        out_specs=pl.BlockSpec((BM, FH), lambda c, i: (c * (N // BM) + i, 0)),
        out_shape=jax.ShapeDtypeStruct((NC * N, FH), jnp.float32),
    )(x, deg_col, w)


def _relu_combine(a0_ref, a1_ref, h0_ref, h1_ref, deg_ref, b_ref):
    deg = deg_ref[...] + 1.0
    dinv = jnp.where(deg > 0, 1.0 / jnp.sqrt(deg), 0.0)
    y0 = dinv * (a0_ref[...] + h0_ref[...])
    y1 = dinv * (a1_ref[...] + h1_ref[...])
    y = jnp.concatenate([y0, y1], axis=1) + b_ref[...]
    return jnp.maximum(y, 0.0), dinv


def _mid_body(a0_ref, a1_ref, h0_ref, h1_ref, deg_ref, b_ref, w_ref, o_ref):
    y, dinv = _relu_combine(a0_ref, a1_ref, h0_ref, h1_ref, deg_ref, b_ref)
    o_ref[...] = jnp.dot(y * dinv, w_ref[...],
                         preferred_element_type=jnp.float32)


def _k_mid(acc, hprime, deg_col, b, w):
    nb = N // BM
    return pl.pallas_call(
        _mid_body,
        grid=(NC, nb),
        in_specs=[
            pl.BlockSpec((BM, FH), lambda c, i: (i, 0)),
            pl.BlockSpec((BM, FH), lambda c, i: (nb + i, 0)),
            pl.BlockSpec((BM, FH), lambda c, i: (i, 0)),
            pl.BlockSpec((BM, FH), lambda c, i: (nb + i, 0)),
            pl.BlockSpec((BM, 1), lambda c, i: (i, 0)),
            pl.BlockSpec((1, F), lambda c, i: (0, 0)),
            pl.BlockSpec((F, FH), lambda c, i: (0, c)),
        ],
        out_specs=pl.BlockSpec((BM, FH), lambda c, i: (c * nb + i, 0)),
        out_shape=jax.ShapeDtypeStruct((NC * N, FH), jnp.float32),
    )(acc, acc, hprime, hprime, deg_col, b, w)


def _tail_body(a0_ref, a1_ref, h0_ref, h1_ref, deg_ref, b_ref,
               w1_ref, b1_ref, w2_ref, b2_ref, w3_ref, b3_ref, o_ref):
    y, _ = _relu_combine(a0_ref, a1_ref, h0_ref, h1_ref, deg_ref, b_ref)
    h = jnp.maximum(jnp.dot(y, w1_ref[...],
                            preferred_element_type=jnp.float32)
                    + b1_ref[...], 0.0)
    h = jnp.maximum(jnp.dot(h, w2_ref[...],
                            preferred_element_type=jnp.float32)
                    + b2_ref[...], 0.0)
    h = jnp.maximum(jnp.dot(h, w3_ref[...],
                            preferred_element_type=jnp.float32)
                    + b3_ref[...], 0.0)
    o_ref[...] = h


def _k_tail(acc, hprime, deg_col, b, w1, b1, w2, b2, w3, b3):
    nb = N // BM
    return pl.pallas_call(
        _tail_body,
        grid=(nb,),
        in_specs=[
            pl.BlockSpec((BM, FH), lambda i: (i, 0)),
            pl.BlockSpec((BM, FH), lambda i: (nb + i, 0)),
            pl.BlockSpec((BM, FH), lambda i: (i, 0)),
            pl.BlockSpec((BM, FH), lambda i: (nb + i, 0)),
            pl.BlockSpec((BM, 1), lambda i: (i, 0)),
            pl.BlockSpec((1, F), lambda i: (0, 0)),
            pl.BlockSpec((256, 256), lambda i: (0, 0)),
            pl.BlockSpec((1, 256), lambda i: (0, 0)),
            pl.BlockSpec((256, 128), lambda i: (0, 0)),
            pl.BlockSpec((1, 128), lambda i: (0, 0)),
            pl.BlockSpec((128, 64), lambda i: (0, 0)),
            pl.BlockSpec((1, 64), lambda i: (0, 0)),
        ],
        out_specs=pl.BlockSpec((BM, 64), lambda i: (i, 0)),
        out_shape=jax.ShapeDtypeStruct((N, 64), jnp.float32),
    )(acc, acc, hprime, hprime, deg_col, b, w1, b1, w2, b2, w3, b3)


BF = 400


def _final_body(a_ref, b_ref, o_ref):
    o_ref[...] = lax.dot_general(
        a_ref[...], b_ref[...],
        (((1,), (1,)), ((), ())),
        preferred_element_type=jnp.float32)


def _final(dis, drg):
    return pl.pallas_call(
        _final_body,
        grid=(N // BF,),
        in_specs=[
            pl.BlockSpec((BF, 64), lambda i: (i, 0)),
            pl.BlockSpec((N, 64), lambda i: (0, 0)),
        ],
        out_specs=pl.BlockSpec((BF, N), lambda i: (i, 0)),
        out_shape=jax.ShapeDtypeStruct((N, N), jnp.float32),
    )(dis, drg)


def kernel(drug_data, drug_edge_index, disease_data, disease_edge_index,
           disease_random, drug_random, Wg1d, bg1d, Wg2d, bg2d, Wg1r, bg1r,
           Wg2r, bg2r, Wl1d, bl1d, Wl2d, bl2d, Wl3d, bl3d, Wl1r, bl1r,
           Wl2r, bl2r, Wl3r, bl3r):
    ds_, dd_ = disease_edge_index[0], disease_edge_index[1]
    rs_, rd_ = drug_edge_index[0], drug_edge_index[1]

    zi = jnp.zeros((E2 - E,), jnp.int32)
    ds2 = jnp.concatenate([ds_, zi]).reshape(NCC, KC)
    dd2 = jnp.concatenate([dd_, zi]).reshape(NCC, KC)
    rs2 = jnp.concatenate([rs_, zi]).reshape(NCC, KC)
    rd2 = jnp.concatenate([rd_, zi]).reshape(NCC, KC)

    ew_d, deg2_d = _sc_prep(disease_data.reshape(-1), ds2, dd2)
    ew_r, deg2_r = _sc_prep(drug_data.reshape(-1), rs2, rd2)

    degcol_d = (deg2_d[:N, :1] + deg2_d[N:, :1])
    degcol_r = (deg2_r[:N, :1] + deg2_r[N:, :1])
    ewd2 = ew_d.reshape(NCC, KC)
    ewr2 = ew_r.reshape(NCC, KC)

    hp_d = _k_pre(disease_random, degcol_d, Wg1d)
    hp_r = _k_pre(drug_random, degcol_r, Wg1r)
    acc_d = _sc_msg(hp_d, ds2, dd2, ewd2)
    acc_r = _sc_msg(hp_r, rs2, rd2, ewr2)
    hp2_d = _k_mid(acc_d, hp_d, degcol_d, bg1d.reshape(1, F), Wg2d)
    hp2_r = _k_mid(acc_r, hp_r, degcol_r, bg1r.reshape(1, F), Wg2r)
    acc2_d = _sc_msg(hp2_d, ds2, dd2, ewd2)
    acc2_r = _sc_msg(hp2_r, rs2, rd2, ewr2)
    dis = _k_tail(acc2_d, hp2_d, degcol_d, bg2d.reshape(1, F),
                  Wl1d, bl1d.reshape(1, 256), Wl2d, bl2d.reshape(1, 128),
                  Wl3d, bl3d.reshape(1, 64))
    drg = _k_tail(acc2_r, hp2_r, degcol_r, bg2r.reshape(1, F),
                  Wl1r, bl1r.reshape(1, 256), Wl2r, bl2r.reshape(1, 128),
                  Wl3r, bl3r.reshape(1, 64))

    return _final(dis, drg)


# speedup vs baseline: 1.1039x; 1.0129x over previous
import functools
import jax
import jax.numpy as jnp
from jax import lax
from jax.experimental import pallas as pl
from jax.experimental.pallas import tpu as pltpu
from jax.experimental.pallas import tpu_sc as plsc

N = 10000
E = 160000
F = 256
FH = 128
NC = 2
NS = 16
L = 16
K = 80
NCHUNK = E // K
DW = 128

def _mesh():
    return plsc.VectorSubcoreMesh(core_axis_name="c", subcore_axis_name="s",
                                  num_cores=NC, num_subcores=NS)


KC = 128
E2 = 163840
NCC = E2 // KC
EPT = E2 // (NC * NS)
CPP = EPT // KC


def _sc_prep_body(sim_hbm, src_hbm, dst_hbm, ew_hbm, deg_hbm,
                  sbuf, dbuf, ibuf, ewbuf, ewb, zbuf, deg_sh):
    c = lax.axis_index("c")
    s = lax.axis_index("s")
    wid = s * NC + c
    rpt = 1000
    zrows = 200

    @pl.loop(0, zrows)
    def _z(i):
        for f in range(DW // L):
            zbuf[i, pl.ds(f * L, L)] = jnp.zeros((L,), jnp.float32)

    @pl.when(s < 10)
    def _():
        @pl.loop(0, rpt // zrows)
        def _zs(j):
            pltpu.sync_copy(zbuf, deg_sh.at[pl.ds(s * rpt + j * zrows,
                                                  zrows)])
    plsc.subcore_barrier()

    @pl.loop(0, K)
    def _ez(i):
        for f in range(DW // L):
            ewb[i, pl.ds(f * L, L)] = jnp.zeros((L,), jnp.float32)

    n_i = (NCHUNK - wid + NC * NS - 1) // (NC * NS)

    @pl.loop(0, n_i)
    def _chunk(i):
        base = (wid + i * NC * NS) * K
        pltpu.sync_copy(src_hbm.at[pl.ds(base, K)], sbuf)
        pltpu.sync_copy(dst_hbm.at[pl.ds(base, K)], dbuf)
        for j in range(K // L):
            sv = sbuf[pl.ds(j * L, L)]
            dv = dbuf[pl.ds(j * L, L)]
            ibuf[pl.ds(j * L, L)] = sv * N + dv
        pltpu.sync_copy(sim_hbm.at[ibuf], ewbuf)
        pltpu.sync_copy(ewbuf, ew_hbm.at[pl.ds(base, K)])
        @pl.loop(0, K // L)
        def _grp(g):
            ewv = ewbuf[pl.ds(g * L, L)]
            for j in range(L):
                w = jnp.take(ewv, jnp.zeros((L,), jnp.int32) + j)
                r = g * L + j
                ewb[r, pl.ds(0, L)] = jnp.zeros((L,), jnp.float32)
                ewb[r, pl.ds(0, L)] = ewb[r, pl.ds(0, L)] + w
        pltpu.sync_copy(ewb, deg_sh.at[dbuf], add=True)

    plsc.subcore_barrier()

    @pl.when(s < 10)
    def _():
        @pl.loop(0, rpt // zrows)
        def _wb(j):
            pltpu.sync_copy(deg_sh.at[pl.ds(s * rpt + j * zrows, zrows)],
                            zbuf)
            pltpu.sync_copy(zbuf,
                            deg_hbm.at[pl.ds(c * N + s * rpt + j * zrows,
                                             zrows)])


def _sc_prep(sim_flat, src, dst):
    kfn = pl.kernel(
        _sc_prep_body,
        out_type=[
            jax.ShapeDtypeStruct((E,), jnp.float32),
            jax.ShapeDtypeStruct((NC * N, DW), jnp.float32),
        ],
        mesh=_mesh(),
        scratch_types=[
            pltpu.VMEM((K,), jnp.int32),
            pltpu.VMEM((K,), jnp.int32),
            pltpu.VMEM((K,), jnp.int32),
            pltpu.VMEM((K,), jnp.float32),
            pltpu.VMEM((K, DW), jnp.float32),
            pltpu.VMEM((200, DW), jnp.float32),
            pltpu.VMEM_SHARED((N, DW), jnp.float32),
        ],
    )
    return kfn(sim_flat, src, dst)


CPT = NCC // NS
GP = 16


def _msg_one_graph(hp_hbm, src_hbm, dst_hbm, ew_hbm, acc_hbm,
                   sidx, didx, ewt, gbufs, zbuf, acc_sh, gsem, ssem, c, s):
    rpt = 1000
    zrows = 40

    @pl.loop(0, zrows)
    def _z(i):
        for f in range(FH // L):
            zbuf[i, pl.ds(f * L, L)] = jnp.zeros((L,), jnp.float32)

    @pl.when(s < 10)
    def _():
        @pl.loop(0, rpt // zrows)
        def _zs(j):
            pltpu.sync_copy(zbuf, acc_sh.at[pl.ds(s * rpt + j * zrows,
                                                  zrows)])
    plsc.subcore_barrier()

    def fire(k, b):
        pltpu.async_copy(hp_hbm.at[sidx.at[k]], gbufs[b], gsem)

    def wait_g(b):
        pltpu.make_async_copy(hp_hbm.at[pl.ds(0, KC)], gbufs[b], gsem).wait()

    def fire_s(k, b):
        pltpu.async_copy(gbufs[b], acc_sh.at[didx.at[k]], ssem, add=True)

    def wait_s(b):
        pltpu.make_async_copy(gbufs[b], acc_sh.at[pl.ds(0, KC)], ssem).wait()

    @pl.loop(0, CPT // GP)
    def _group(grp):
        gb = s * CPT + grp * GP
        pltpu.sync_copy(src_hbm.at[pl.ds(gb, GP)], sidx)
        pltpu.sync_copy(dst_hbm.at[pl.ds(gb, GP)], didx)
        pltpu.sync_copy(ew_hbm.at[pl.ds(gb, GP)], ewt)

        @pl.loop(0, GP)
        def _adj(r):
            for g in range(KC // L):
                sidx[r, pl.ds(g * L, L)] = sidx[r, pl.ds(g * L, L)] + c * N

        fire(0, 0)
        for k in range(GP):
            wait_g(k % 2)
            if k < GP - 1:
                if k >= 1:
                    wait_s((k + 1) % 2)
                fire(k + 1, (k + 1) % 2)
            g = gbufs[k % 2]

            @pl.loop(0, KC // L)
            def _scale(gi):
                ewv = ewt[k, pl.ds(gi * L, L)]
                ws = [jnp.take(ewv, jnp.zeros((L,), jnp.int32) + j)
                      for j in range(L)]
                for j in range(L):
                    e = gi * L + j
                    for f in range(FH // L):
                        g[e, pl.ds(f * L, L)] = (g[e, pl.ds(f * L, L)]
                                                 * ws[j])

            fire_s(k, k % 2)

        wait_s(0)
        wait_s(1)

    plsc.subcore_barrier()

    @pl.when(s < 10)
    def _():
        @pl.loop(0, rpt // zrows)
        def _wb(j):
            pltpu.sync_copy(acc_sh.at[pl.ds(s * rpt + j * zrows, zrows)],
                            zbuf)
            pltpu.sync_copy(zbuf,
                            acc_hbm.at[pl.ds(c * N + s * rpt + j * zrows,
                                             zrows)])


def _sc_msg_body(hp_hbm, src_hbm, dst_hbm, ew_hbm, acc_hbm,
                 sidx, didx, ewt, gbuf0, gbuf1, zbuf, acc_sh, gsem, ssem):
    c = lax.axis_index("c")
    s = lax.axis_index("s")
    gbufs = (gbuf0, gbuf1)
    _msg_one_graph(hp_hbm, src_hbm, dst_hbm, ew_hbm, acc_hbm, sidx, didx,
                   ewt, gbufs, zbuf, acc_sh, gsem, ssem, c, s)


def _sc_msg(hprime, src2d, dst2d, ew2d):
    kfn = pl.kernel(
        _sc_msg_body,
        out_type=jax.ShapeDtypeStruct((NC * N, FH), jnp.float32),
        mesh=_mesh(),
        scratch_types=[
            pltpu.VMEM((GP, KC), jnp.int32),
            pltpu.VMEM((GP, KC), jnp.int32),
            pltpu.VMEM((GP, KC), jnp.float32),
            pltpu.VMEM((KC, FH), jnp.float32),
            pltpu.VMEM((KC, FH), jnp.float32),
            pltpu.VMEM((40, FH), jnp.float32),
            pltpu.VMEM_SHARED((N, FH), jnp.float32),
            pltpu.SemaphoreType.DMA,
            pltpu.SemaphoreType.DMA,
        ],
    )
    return kfn(hprime, src2d, dst2d, ew2d)


BM = 400


def _pre_body(x_ref, deg_ref, w_ref, o_ref):
    deg = deg_ref[...] + 1.0
    dinv = jnp.where(deg > 0, 1.0 / jnp.sqrt(deg), 0.0)
    o_ref[...] = jnp.dot(x_ref[...] * dinv, w_ref[...],
                         preferred_element_type=jnp.float32)


def _k_pre(x, deg_col, w):
    fin = x.shape[1]
    return pl.pallas_call(
        _pre_body,
        grid=(NC, N // BM),
        in_specs=[
            pl.BlockSpec((BM, fin), lambda c, i: (i, 0)),
            pl.BlockSpec((BM, 1), lambda c, i: (i, 0)),
            pl.BlockSpec((fin, FH), lambda c, i: (0, c)),
        ],
        out_specs=pl.BlockSpec((BM, FH), lambda c, i: (c * (N // BM) + i, 0)),
        out_shape=jax.ShapeDtypeStruct((NC * N, FH), jnp.float32),
    )(x, deg_col, w)


def _relu_combine(a0_ref, a1_ref, h0_ref, h1_ref, deg_ref, b_ref):
    deg = deg_ref[...] + 1.0
    dinv = jnp.where(deg > 0, 1.0 / jnp.sqrt(deg), 0.0)
    y0 = dinv * (a0_ref[...] + h0_ref[...])
    y1 = dinv * (a1_ref[...] + h1_ref[...])
    y = jnp.concatenate([y0, y1], axis=1) + b_ref[...]
    return jnp.maximum(y, 0.0), dinv


def _mid_body(a0_ref, a1_ref, h0_ref, h1_ref, deg_ref, b_ref, w_ref, o_ref):
    y, dinv = _relu_combine(a0_ref, a1_ref, h0_ref, h1_ref, deg_ref, b_ref)
    o_ref[...] = jnp.dot(y * dinv, w_ref[...],
                         preferred_element_type=jnp.float32)


def _k_mid(acc, hprime, deg_col, b, w):
    nb = N // BM
    return pl.pallas_call(
        _mid_body,
        grid=(NC, nb),
        in_specs=[
            pl.BlockSpec((BM, FH), lambda c, i: (i, 0)),
            pl.BlockSpec((BM, FH), lambda c, i: (nb + i, 0)),
            pl.BlockSpec((BM, FH), lambda c, i: (i, 0)),
            pl.BlockSpec((BM, FH), lambda c, i: (nb + i, 0)),
            pl.BlockSpec((BM, 1), lambda c, i: (i, 0)),
            pl.BlockSpec((1, F), lambda c, i: (0, 0)),
            pl.BlockSpec((F, FH), lambda c, i: (0, c)),
        ],
        out_specs=pl.BlockSpec((BM, FH), lambda c, i: (c * nb + i, 0)),
        out_shape=jax.ShapeDtypeStruct((NC * N, FH), jnp.float32),
    )(acc, acc, hprime, hprime, deg_col, b, w)


def _tail_body(a0_ref, a1_ref, h0_ref, h1_ref, deg_ref, b_ref,
               w1_ref, b1_ref, w2_ref, b2_ref, w3_ref, b3_ref, o_ref):
    y, _ = _relu_combine(a0_ref, a1_ref, h0_ref, h1_ref, deg_ref, b_ref)
    h = jnp.maximum(jnp.dot(y, w1_ref[...],
                            preferred_element_type=jnp.float32)
                    + b1_ref[...], 0.0)
    h = jnp.maximum(jnp.dot(h, w2_ref[...],
                            preferred_element_type=jnp.float32)
                    + b2_ref[...], 0.0)
    h = jnp.maximum(jnp.dot(h, w3_ref[...],
                            preferred_element_type=jnp.float32)
                    + b3_ref[...], 0.0)
    o_ref[...] = h


def _k_tail(acc, hprime, deg_col, b, w1, b1, w2, b2, w3, b3):
    nb = N // BM
    return pl.pallas_call(
        _tail_body,
        grid=(nb,),
        in_specs=[
            pl.BlockSpec((BM, FH), lambda i: (i, 0)),
            pl.BlockSpec((BM, FH), lambda i: (nb + i, 0)),
            pl.BlockSpec((BM, FH), lambda i: (i, 0)),
            pl.BlockSpec((BM, FH), lambda i: (nb + i, 0)),
            pl.BlockSpec((BM, 1), lambda i: (i, 0)),
            pl.BlockSpec((1, F), lambda i: (0, 0)),
            pl.BlockSpec((256, 256), lambda i: (0, 0)),
            pl.BlockSpec((1, 256), lambda i: (0, 0)),
            pl.BlockSpec((256, 128), lambda i: (0, 0)),
            pl.BlockSpec((1, 128), lambda i: (0, 0)),
            pl.BlockSpec((128, 64), lambda i: (0, 0)),
            pl.BlockSpec((1, 64), lambda i: (0, 0)),
        ],
        out_specs=pl.BlockSpec((BM, 64), lambda i: (i, 0)),
        out_shape=jax.ShapeDtypeStruct((N, 64), jnp.float32),
    )(acc, acc, hprime, hprime, deg_col, b, w1, b1, w2, b2, w3, b3)


BF = 400


def _final_body(a_ref, b_ref, o_ref):
    o_ref[...] = lax.dot_general(
        a_ref[...], b_ref[...],
        (((1,), (1,)), ((), ())),
        preferred_element_type=jnp.float32)


def _final(dis, drg):
    return pl.pallas_call(
        _final_body,
        grid=(N // BF,),
        in_specs=[
            pl.BlockSpec((BF, 64), lambda i: (i, 0)),
            pl.BlockSpec((N, 64), lambda i: (0, 0)),
        ],
        out_specs=pl.BlockSpec((BF, N), lambda i: (i, 0)),
        out_shape=jax.ShapeDtypeStruct((N, N), jnp.float32),
    )(dis, drg)


def kernel(drug_data, drug_edge_index, disease_data, disease_edge_index,
           disease_random, drug_random, Wg1d, bg1d, Wg2d, bg2d, Wg1r, bg1r,
           Wg2r, bg2r, Wl1d, bl1d, Wl2d, bl2d, Wl3d, bl3d, Wl1r, bl1r,
           Wl2r, bl2r, Wl3r, bl3r):
    ds_, dd_ = disease_edge_index[0], disease_edge_index[1]
    rs_, rd_ = drug_edge_index[0], drug_edge_index[1]

    zi = jnp.zeros((E2 - E,), jnp.int32)
    ds2 = jnp.concatenate([ds_, zi]).reshape(NCC, KC)
    dd2 = jnp.concatenate([dd_, zi]).reshape(NCC, KC)
    rs2 = jnp.concatenate([rs_, zi]).reshape(NCC, KC)
    rd2 = jnp.concatenate([rd_, zi]).reshape(NCC, KC)

    ew_d, deg2_d = _sc_prep(disease_data.reshape(-1), ds_, dd_)
    ew_r, deg2_r = _sc_prep(drug_data.reshape(-1), rs_, rd_)

    degcol_d = (deg2_d[:N, :1] + deg2_d[N:, :1])
    degcol_r = (deg2_r[:N, :1] + deg2_r[N:, :1])
    zf = jnp.zeros((E2 - E,), jnp.float32)
    ewd2 = jnp.concatenate([ew_d, zf]).reshape(NCC, KC)
    ewr2 = jnp.concatenate([ew_r, zf]).reshape(NCC, KC)

    hp_d = _k_pre(disease_random, degcol_d, Wg1d)
    hp_r = _k_pre(drug_random, degcol_r, Wg1r)
    acc_d = _sc_msg(hp_d, ds2, dd2, ewd2)
    acc_r = _sc_msg(hp_r, rs2, rd2, ewr2)
    hp2_d = _k_mid(acc_d, hp_d, degcol_d, bg1d.reshape(1, F), Wg2d)
    hp2_r = _k_mid(acc_r, hp_r, degcol_r, bg1r.reshape(1, F), Wg2r)
    acc2_d = _sc_msg(hp2_d, ds2, dd2, ewd2)
    acc2_r = _sc_msg(hp2_r, rs2, rd2, ewr2)
    dis = _k_tail(acc2_d, hp2_d, degcol_d, bg2d.reshape(1, F),
                  Wl1d, bl1d.reshape(1, 256), Wl2d, bl2d.reshape(1, 128),
                  Wl3d, bl3d.reshape(1, 64))
    drg = _k_tail(acc2_r, hp2_r, degcol_r, bg2r.reshape(1, F),
                  Wl1r, bl1r.reshape(1, 256), Wl2r, bl2r.reshape(1, 128),
                  Wl3r, bl3r.reshape(1, 64))

    return _final(dis, drg)
